# baseline (speedup 1.0000x reference)
import functools

import jax
import jax.numpy as jnp
from jax import lax
from jax.experimental import pallas as pl
from jax.experimental.pallas import tpu as pltpu
from jax.experimental.pallas import tpu_sc as plsc

N = 4096
E = 16384
IN_F = 256
OUT_F = 128

_RB2 = 1024

def _k_sup_agg(p_ref, x_ref, w_ref, adj_ref, prod_ref, recip_ref, sm_s):
    @pl.when(pl.program_id(0) == 0)
    def _():
        p2 = 2.0 * jax.nn.sigmoid(p_ref[...])
        support = jnp.dot(x_ref[...], w_ref[...],
                          preferred_element_type=jnp.float32)
        e = support * p2
        sm = jnp.exp(e - jnp.max(e))
        sm_s[...] = sm
        prod_ref[...] = support * sm

    agg = jnp.dot(adj_ref[...], sm_s[...],
                  preferred_element_type=jnp.float32)
    d = agg + 1e-6
    r = 1.0 / d
    r = r * (2.0 - d * r)
    recip_ref[...] = r * (2.0 - d * r)


def _sup_agg(p, x, w, adj):
    grid = (N // _RB2,)
    return pl.pallas_call(
        _k_sup_agg,
        grid=grid,
        in_specs=[
            pl.BlockSpec((1, 1), lambda i: (0, 0)),
            pl.BlockSpec((N, IN_F), lambda i: (0, 0)),
            pl.BlockSpec((IN_F, OUT_F), lambda i: (0, 0)),
            pl.BlockSpec((_RB2, N), lambda i: (i, 0)),
        ],
        out_specs=(
            pl.BlockSpec((N, OUT_F), lambda i: (0, 0)),
            pl.BlockSpec((_RB2, OUT_F), lambda i: (i, 0)),
        ),
        out_shape=(
            jax.ShapeDtypeStruct((N, OUT_F), jnp.float32),
            jax.ShapeDtypeStruct((N, OUT_F), jnp.float32),
        ),
        scratch_shapes=[pltpu.VMEM((N, OUT_F), jnp.float32)],
        compiler_params=pltpu.CompilerParams(
            dimension_semantics=("arbitrary",)),
    )(p, x, w, adj)


_NC = 2
_NS = 16
_NW = _NC * _NS
_EPW = E // _NW
_CHUNK = 128
_NCHUNK = _EPW // _CHUNK


def _gather_mul(prod, recip, edge):
    mesh = plsc.VectorSubcoreMesh(core_axis_name="c", subcore_axis_name="s")

    @functools.partial(
        pl.kernel,
        mesh=mesh,
        out_type=jax.ShapeDtypeStruct((E, OUT_F), jnp.float32),
        scratch_types=[
            pltpu.VMEM((_EPW,), jnp.int32),
            pltpu.VMEM((_EPW,), jnp.int32),
            pltpu.VMEM((2, _CHUNK, OUT_F), jnp.float32),
            pltpu.VMEM((2, _CHUNK, OUT_F), jnp.float32),
            pltpu.SemaphoreType.DMA,
            pltpu.SemaphoreType.DMA,
            pltpu.SemaphoreType.DMA,
            pltpu.SemaphoreType.DMA,
            pltpu.SemaphoreType.DMA,
            pltpu.SemaphoreType.DMA,
        ],
    )
    def k(prod_hbm, recip_hbm, edge_hbm, msg_hbm,
          idx1_v, idx0_v, r1_v, r0_v,
          g1a, g1b, g0a, g0b, wa, wb):
        wid = lax.axis_index("s") * _NC + lax.axis_index("c")
        base = wid * _EPW
        gsems1 = (g1a, g1b)
        gsems0 = (g0a, g0b)
        wsems = (wa, wb)

        pltpu.sync_copy(edge_hbm.at[1, pl.ds(base, _EPW)], idx1_v)
        pltpu.sync_copy(edge_hbm.at[0, pl.ds(base, _EPW)], idx0_v)

        def start(c, slot):
            isl = pl.ds(c * _CHUNK, _CHUNK)
            cp1 = pltpu.async_copy(prod_hbm.at[idx1_v.at[isl]], r1_v.at[slot],
                                   gsems1[slot])
            cp0 = pltpu.async_copy(recip_hbm.at[idx0_v.at[isl]], r0_v.at[slot],
                                   gsems0[slot])
            return cp1, cp0

        def mul_rows(slot):
            def rowbody(r, carry):
                for cvec in range(OUT_F // 16):
                    sl = pl.ds(cvec * 16, 16)
                    r1_v[slot, r, sl] = r1_v[slot, r, sl] * r0_v[slot, r, sl]
                return carry
            lax.fori_loop(0, _CHUNK, rowbody, 0)

        scat = [None, None]
        cur = start(0, 0)
        for c in range(_NCHUNK):
            slot = c % 2
            nxt = None
            if c + 1 < _NCHUNK:
                if scat[1 - slot] is not None:
                    scat[1 - slot].wait()
                nxt = start(c + 1, 1 - slot)
            cur[0].wait()
            cur[1].wait()
            mul_rows(slot)
            off = base + c * _CHUNK
            scat[slot] = pltpu.async_copy(
                r1_v.at[slot], msg_hbm.at[pl.ds(off, _CHUNK)], wsems[slot])
            cur = nxt
        for s in (0, 1):
            if scat[s] is not None:
                scat[s].wait()

    return k(prod, recip, edge)


_RB3 = 64

def _k_out(t_ref, msg_ref, b_ref, out_ref):
    out_ref[...] = jnp.dot(t_ref[...], msg_ref[...],
                           preferred_element_type=jnp.float32) + b_ref[...]


def _final(T, msg, bias):
    grid = (N // _RB3,)
    return pl.pallas_call(
        _k_out,
        grid=grid,
        in_specs=[
            pl.BlockSpec((_RB3, E), lambda i: (i, 0)),
            pl.BlockSpec((E, OUT_F), lambda i: (0, 0)),
            pl.BlockSpec((1, OUT_F), lambda i: (0, 0)),
        ],
        out_specs=pl.BlockSpec((_RB3, OUT_F), lambda i: (i, 0)),
        out_shape=jax.ShapeDtypeStruct((N, OUT_F), jnp.float32),
        compiler_params=pltpu.CompilerParams(
            dimension_semantics=("arbitrary",)),
    )(T, msg, bias)


def kernel(input, T, adj, edge, p, weight, bias):
    p11 = p.reshape(1, 1)
    prod, recip = _sup_agg(p11, input, weight, adj)
    msg = _gather_mul(prod, recip, edge)
    return _final(T, msg, bias.reshape(1, OUT_F))


# speedup vs baseline: 1.1454x; 1.1454x over previous
import functools

import jax
import jax.numpy as jnp
from jax import lax
from jax.experimental import pallas as pl
from jax.experimental.pallas import tpu as pltpu
from jax.experimental.pallas import tpu_sc as plsc

N = 4096
E = 16384
IN_F = 256
OUT_F = 128

_RB2 = 1024

def _k_sup_agg(p_ref, x_ref, w_ref, adj_ref, prod_ref, recip_ref, sm_s):
    @pl.when(pl.program_id(0) == 0)
    def _():
        p2 = 2.0 * jax.nn.sigmoid(p_ref[...])
        support = jnp.dot(x_ref[...], w_ref[...],
                          preferred_element_type=jnp.float32)
        e = support * p2
        sm = jnp.exp(e - jnp.max(e))
        sm_s[...] = sm
        prod_ref[...] = support * sm

    agg = jnp.dot(adj_ref[...], sm_s[...],
                  preferred_element_type=jnp.float32)
    d = agg + 1e-6
    r = 1.0 / d
    r = r * (2.0 - d * r)
    recip_ref[...] = r * (2.0 - d * r)


def _sup_agg(p, x, w, adj):
    grid = (N // _RB2,)
    return pl.pallas_call(
        _k_sup_agg,
        grid=grid,
        in_specs=[
            pl.BlockSpec((1, 1), lambda i: (0, 0)),
            pl.BlockSpec((N, IN_F), lambda i: (0, 0)),
            pl.BlockSpec((IN_F, OUT_F), lambda i: (0, 0)),
            pl.BlockSpec((_RB2, N), lambda i: (i, 0)),
        ],
        out_specs=(
            pl.BlockSpec((N, OUT_F), lambda i: (0, 0)),
            pl.BlockSpec((_RB2, OUT_F), lambda i: (i, 0)),
        ),
        out_shape=(
            jax.ShapeDtypeStruct((N, OUT_F), jnp.float32),
            jax.ShapeDtypeStruct((N, OUT_F), jnp.float32),
        ),
        scratch_shapes=[pltpu.VMEM((N, OUT_F), jnp.float32)],
        compiler_params=pltpu.CompilerParams(
            dimension_semantics=("arbitrary",)),
    )(p, x, w, adj)


_NC = 2
_NS = 16
_NW = _NC * _NS
_EPW = E // _NW
_CHUNK = 128
_NCHUNK = _EPW // _CHUNK
_RING = 3


def _gather_mul(prod, recip, edge):
    mesh = plsc.VectorSubcoreMesh(core_axis_name="c", subcore_axis_name="s")

    @functools.partial(
        pl.kernel,
        mesh=mesh,
        out_type=jax.ShapeDtypeStruct((E, OUT_F), jnp.float32),
        scratch_types=[
            pltpu.VMEM((_EPW,), jnp.int32),
            pltpu.VMEM((_EPW,), jnp.int32),
            pltpu.VMEM((_RING, _CHUNK, OUT_F), jnp.float32),
            pltpu.VMEM((_RING, _CHUNK, OUT_F), jnp.float32),
        ] + [pltpu.SemaphoreType.DMA] * (3 * _RING),
    )
    def k(prod_hbm, recip_hbm, edge_hbm, msg_hbm,
          idx1_v, idx0_v, r1_v, r0_v, *sems):
        wid = lax.axis_index("s") * _NC + lax.axis_index("c")
        base = wid * _EPW
        gsems1 = sems[:_RING]
        gsems0 = sems[_RING:2 * _RING]
        wsems = sems[2 * _RING:]

        pltpu.sync_copy(edge_hbm.at[1, pl.ds(base, _EPW)], idx1_v)
        pltpu.sync_copy(edge_hbm.at[0, pl.ds(base, _EPW)], idx0_v)

        def issue(c, slot):
            isl = pl.ds(c * _CHUNK, _CHUNK)
            cp1 = pltpu.async_copy(prod_hbm.at[idx1_v.at[isl]], r1_v.at[slot],
                                   gsems1[slot])
            cp0 = pltpu.async_copy(recip_hbm.at[idx0_v.at[isl]], r0_v.at[slot],
                                   gsems0[slot])
            return cp1, cp0

        def mul_rows(slot):
            def rowbody(r, carry):
                for cvec in range(OUT_F // 16):
                    sl = pl.ds(cvec * 16, 16)
                    r1_v[slot, r, sl] = r1_v[slot, r, sl] * r0_v[slot, r, sl]
                return carry
            lax.fori_loop(0, _CHUNK, rowbody, 0)

        pend = [None] * _NCHUNK
        scat = [None] * _RING
        for c in range(min(_RING, _NCHUNK)):
            pend[c] = issue(c, c)
        for c in range(_NCHUNK):
            slot = c % _RING
            pend[c][0].wait()
            pend[c][1].wait()
            mul_rows(slot)
            off = base + c * _CHUNK
            scat[slot] = pltpu.async_copy(
                r1_v.at[slot], msg_hbm.at[pl.ds(off, _CHUNK)], wsems[slot])
            n = c + _RING
            if n < _NCHUNK:
                nslot = n % _RING
                scat[nslot].wait()
                pend[n] = issue(n, nslot)
        for s in scat:
            if s is not None:
                s.wait()

    return k(prod, recip, edge)


_RB3 = 128

def _k_out(t_ref, msg_ref, b_ref, out_ref):
    out_ref[...] = jnp.dot(t_ref[...], msg_ref[...],
                           preferred_element_type=jnp.float32) + b_ref[...]


def _final(T, msg, bias):
    grid = (N // _RB3,)
    return pl.pallas_call(
        _k_out,
        grid=grid,
        in_specs=[
            pl.BlockSpec((_RB3, E), lambda i: (i, 0)),
            pl.BlockSpec((E, OUT_F), lambda i: (0, 0)),
            pl.BlockSpec((1, OUT_F), lambda i: (0, 0)),
        ],
        out_specs=pl.BlockSpec((_RB3, OUT_F), lambda i: (i, 0)),
        out_shape=jax.ShapeDtypeStruct((N, OUT_F), jnp.float32),
        compiler_params=pltpu.CompilerParams(
            dimension_semantics=("arbitrary",)),
    )(T, msg, bias)


def kernel(input, T, adj, edge, p, weight, bias):
    p11 = p.reshape(1, 1)
    prod, recip = _sup_agg(p11, input, weight, adj)
    msg = _gather_mul(prod, recip, edge)
    return _final(T, msg, bias.reshape(1, OUT_F))


# speedup vs baseline: 1.1497x; 1.0037x over previous
import functools

import jax
import jax.numpy as jnp
from jax import lax
from jax.experimental import pallas as pl
from jax.experimental.pallas import tpu as pltpu
from jax.experimental.pallas import tpu_sc as plsc

N = 4096
E = 16384
IN_F = 256
OUT_F = 128

_RB2 = 512

def _k_sup_agg(p_ref, x_ref, w_ref, adj_ref, prod_ref, recip_ref, sm_s):
    @pl.when(pl.program_id(0) == 0)
    def _():
        p2 = 2.0 * jax.nn.sigmoid(p_ref[...])
        support = jnp.dot(x_ref[...], w_ref[...],
                          preferred_element_type=jnp.float32)
        e = support * p2
        sm = jnp.exp(e - jnp.max(e))
        sm_s[...] = sm
        prod_ref[...] = support * sm

    agg = jnp.dot(adj_ref[...], sm_s[...],
                  preferred_element_type=jnp.float32)
    d = agg + 1e-6
    r = 1.0 / d
    r = r * (2.0 - d * r)
    recip_ref[...] = r * (2.0 - d * r)


def _sup_agg(p, x, w, adj):
    grid = (N // _RB2,)
    return pl.pallas_call(
        _k_sup_agg,
        grid=grid,
        in_specs=[
            pl.BlockSpec((1, 1), lambda i: (0, 0)),
            pl.BlockSpec((N, IN_F), lambda i: (0, 0)),
            pl.BlockSpec((IN_F, OUT_F), lambda i: (0, 0)),
            pl.BlockSpec((_RB2, N), lambda i: (i, 0)),
        ],
        out_specs=(
            pl.BlockSpec((N, OUT_F), lambda i: (0, 0)),
            pl.BlockSpec((_RB2, OUT_F), lambda i: (i, 0)),
        ),
        out_shape=(
            jax.ShapeDtypeStruct((N, OUT_F), jnp.float32),
            jax.ShapeDtypeStruct((N, OUT_F), jnp.float32),
        ),
        scratch_shapes=[pltpu.VMEM((N, OUT_F), jnp.float32)],
        compiler_params=pltpu.CompilerParams(
            dimension_semantics=("arbitrary",)),
    )(p, x, w, adj)


_NC = 2
_NS = 16
_NW = _NC * _NS
_EPW = E // _NW
_CHUNK = 128
_NCHUNK = _EPW // _CHUNK
_RING = 3


def _gather_mul(prod, recip, edge):
    mesh = plsc.VectorSubcoreMesh(core_axis_name="c", subcore_axis_name="s")

    @functools.partial(
        pl.kernel,
        mesh=mesh,
        out_type=jax.ShapeDtypeStruct((E, OUT_F), jnp.float32),
        scratch_types=[
            pltpu.VMEM((_EPW,), jnp.int32),
            pltpu.VMEM((_EPW,), jnp.int32),
            pltpu.VMEM((_RING, _CHUNK, OUT_F), jnp.float32),
            pltpu.VMEM((_RING, _CHUNK, OUT_F), jnp.float32),
        ] + [pltpu.SemaphoreType.DMA] * (3 * _RING),
    )
    def k(prod_hbm, recip_hbm, edge_hbm, msg_hbm,
          idx1_v, idx0_v, r1_v, r0_v, *sems):
        wid = lax.axis_index("s") * _NC + lax.axis_index("c")
        base = wid * _EPW
        gsems1 = sems[:_RING]
        gsems0 = sems[_RING:2 * _RING]
        wsems = sems[2 * _RING:]

        pltpu.sync_copy(edge_hbm.at[1, pl.ds(base, _EPW)], idx1_v)
        pltpu.sync_copy(edge_hbm.at[0, pl.ds(base, _EPW)], idx0_v)

        def issue(c, slot):
            isl = pl.ds(c * _CHUNK, _CHUNK)
            cp1 = pltpu.async_copy(prod_hbm.at[idx1_v.at[isl]], r1_v.at[slot],
                                   gsems1[slot])
            cp0 = pltpu.async_copy(recip_hbm.at[idx0_v.at[isl]], r0_v.at[slot],
                                   gsems0[slot])
            return cp1, cp0

        def mul_rows(slot):
            def rowbody(r, carry):
                for cvec in range(OUT_F // 16):
                    sl = pl.ds(cvec * 16, 16)
                    r1_v[slot, r, sl] = r1_v[slot, r, sl] * r0_v[slot, r, sl]
                return carry
            lax.fori_loop(0, _CHUNK, rowbody, 0)

        pend = [None] * _NCHUNK
        scat = [None] * _RING
        for c in range(min(_RING, _NCHUNK)):
            pend[c] = issue(c, c)
        for c in range(_NCHUNK):
            slot = c % _RING
            pend[c][0].wait()
            pend[c][1].wait()
            mul_rows(slot)
            off = base + c * _CHUNK
            scat[slot] = pltpu.async_copy(
                r1_v.at[slot], msg_hbm.at[pl.ds(off, _CHUNK)], wsems[slot])
            n = c + _RING
            if n < _NCHUNK:
                nslot = n % _RING
                scat[nslot].wait()
                pend[n] = issue(n, nslot)
        for s in scat:
            if s is not None:
                s.wait()

    return k(prod, recip, edge)


_RB3 = 128

def _k_out(t_ref, msg_ref, b_ref, out_ref):
    out_ref[...] = jnp.dot(t_ref[...], msg_ref[...],
                           preferred_element_type=jnp.float32) + b_ref[...]


def _final(T, msg, bias):
    grid = (N // _RB3,)
    return pl.pallas_call(
        _k_out,
        grid=grid,
        in_specs=[
            pl.BlockSpec((_RB3, E), lambda i: (i, 0)),
            pl.BlockSpec((E, OUT_F), lambda i: (0, 0)),
            pl.BlockSpec((1, OUT_F), lambda i: (0, 0)),
        ],
        out_specs=pl.BlockSpec((_RB3, OUT_F), lambda i: (i, 0)),
        out_shape=jax.ShapeDtypeStruct((N, OUT_F), jnp.float32),
        compiler_params=pltpu.CompilerParams(
            dimension_semantics=("arbitrary",)),
    )(T, msg, bias)


def kernel(input, T, adj, edge, p, weight, bias):
    p11 = p.reshape(1, 1)
    prod, recip = _sup_agg(p11, input, weight, adj)
    msg = _gather_mul(prod, recip, edge)
    return _final(T, msg, bias.reshape(1, OUT_F))
